# baseline (speedup 1.0000x reference)
import functools

import jax
import jax.numpy as jnp
from jax import lax
from jax.experimental import pallas as pl
from jax.experimental.pallas import tpu as pltpu
from jax.experimental.pallas import tpu_sc as plsc

ROWS, COLS = 16384, 128
N = ROWS * COLS
TABLE = 1000000
NC, NS, L = 2, 16, 16
NW = NC * NS
PER_W = N // NW
CH = 8192
NCH = PER_W // CH
NSTAGE = TABLE // CH
STAGE_TAIL = TABLE - NSTAGE * CH

_mesh = plsc.VectorSubcoreMesh(core_axis_name="c", subcore_axis_name="s")


@functools.partial(
    pl.kernel,
    mesh=_mesh,
    out_type=jax.ShapeDtypeStruct((N,), jnp.float32),
    scratch_types=[
        pltpu.VMEM_SHARED((TABLE,), jnp.float32),
        pltpu.VMEM((CH,), jnp.float32), pltpu.VMEM((CH,), jnp.float32),
        pltpu.VMEM((CH,), jnp.int32), pltpu.VMEM((CH,), jnp.int32),
        pltpu.VMEM((CH,), jnp.float32), pltpu.VMEM((CH,), jnp.float32),
        pltpu.SemaphoreType.DMA, pltpu.SemaphoreType.DMA,
        pltpu.SemaphoreType.DMA, pltpu.SemaphoreType.DMA,
        pltpu.SemaphoreType.DMA, pltpu.SemaphoreType.DMA,
        pltpu.SemaphoreType.DMA, pltpu.SemaphoreType.DMA,
    ],
)
def _lookup(soft_hbm, table_hbm, out_hbm, spt, soft0, soft1, idx0, idx1,
            got0, got1, lsem0, lsem1, gsem0, gsem1, ssem0, ssem1, stsem,
            bsem):
    soft = (soft0, soft1)
    idx = (idx0, idx1)
    got = (got0, got1)
    lsem = (lsem0, lsem1)
    gsem = (gsem0, gsem1)
    ssem = (ssem0, ssem1)

    sid = lax.axis_index("s")
    wid = sid * NC + lax.axis_index("c")
    base = wid * PER_W

    n_rounds = (NSTAGE + NS - 1) // NS

    def st_off(j):
        return j * NS * CH + sid * CH

    def stA(j):
        return pltpu.make_async_copy(
            table_hbm.at[pl.ds(st_off(j), CH)], got[j & 1], stsem)

    def stB(j):
        return pltpu.make_async_copy(
            got[j & 1], spt.at[pl.ds(st_off(j), CH)], bsem)

    def load(c, s):
        return pltpu.make_async_copy(
            soft_hbm.at[pl.ds(base + c * CH, CH)], soft[s], lsem[s])

    def gather(s):
        return pltpu.make_async_copy(spt.at[idx[s]], got[s], gsem[s])

    def store(c, s):
        return pltpu.make_async_copy(
            got[s], out_hbm.at[pl.ds(base + c * CH, CH)], ssem[s])

    def quant_span(s, lo, hi):
        src, dst = soft[s], idx[s]

        @plsc.parallel_loop(lo, hi, step=L, unroll=8)
        def _body(i):
            v = src[pl.ds(i, L)]
            v = jnp.minimum(jnp.maximum(v, -0.999), 0.999)
            dst[pl.ds(i, L)] = (
                (v + 1.0) / 2.0 * float(TABLE)).astype(jnp.int32)

    def quant(s):
        quant_span(s, 0, CH)

    QB = (CH // L) // 4

    def quant_block(b):
        s = b // 4
        base_i = (b % 4) * QB
        quant_span(s, base_i * L, (base_i + QB) * L)

    load(0, 0).start()
    load(1, 1).start()

    def pred(j):
        return st_off(j) + CH <= TABLE

    @pl.when(pred(0))
    def _():
        stA(0).start()
    @pl.when(pred(1))
    def _():
        stA(1).start()
    load(0, 0).wait()
    for j in range(n_rounds):
        if j >= 2:
            @pl.when(pred(j - 2))
            def _(j=j):
                stB(j - 2).wait()
            @pl.when(pred(j))
            def _(j=j):
                stA(j).start()
        if j == 4:
            load(1, 1).wait()
        quant_block(j)
        if j == 3:
            load(2, 0).start()
        if j == 7:
            load(3, 1).start()
        @pl.when(pred(j))
        def _(j=j):
            stA(j).wait()
            stB(j).start()
    @pl.when(pred(n_rounds - 2))
    def _():
        stB(n_rounds - 2).wait()
    @pl.when(pred(n_rounds - 1))
    def _():
        stB(n_rounds - 1).wait()
    @pl.when(sid == NS - 1)
    def _():
        pltpu.sync_copy(table_hbm.at[pl.ds(NSTAGE * CH, STAGE_TAIL)],
                        got0.at[pl.ds(0, STAGE_TAIL)])
        pltpu.sync_copy(got0.at[pl.ds(0, STAGE_TAIL)],
                        spt.at[pl.ds(NSTAGE * CH, STAGE_TAIL)])
    plsc.subcore_barrier()

    for c in range(NCH):
        s = c & 1
        p = s ^ 1
        if c >= 2:
            load(c, s).wait()
            quant(s)
        if 2 <= c and c + 2 < NCH:
            load(c + 2, s).start()
        if c >= 2:
            store(c - 2, s).wait()
        gather(s).start()
        if c >= 1:
            gather(p).wait()
            store(c - 1, p).start()
    last = NCH - 1
    sl = last & 1
    gather(sl).wait()
    store(last, sl).start()
    store(last - 1, sl ^ 1).wait()
    store(last, sl).wait()


def kernel(x, pallette, indices):
    out = _lookup(indices.reshape(-1), pallette.reshape(-1))
    return out.reshape(ROWS, COLS)


# speedup vs baseline: 1.0083x; 1.0083x over previous
import functools

import jax
import jax.numpy as jnp
from jax import lax
from jax.experimental import pallas as pl
from jax.experimental.pallas import tpu as pltpu
from jax.experimental.pallas import tpu_sc as plsc

ROWS, COLS = 16384, 128
N = ROWS * COLS
TABLE = 1000000
NC, NS, L = 2, 16, 16
NW = NC * NS
PER_W = N // NW
CH = 8192
NCH = PER_W // CH
NSTAGE = TABLE // CH
STAGE_TAIL = TABLE - NSTAGE * CH

_mesh = plsc.VectorSubcoreMesh(core_axis_name="c", subcore_axis_name="s")


@functools.partial(
    pl.kernel,
    mesh=_mesh,
    out_type=jax.ShapeDtypeStruct((N,), jnp.float32),
    scratch_types=[
        pltpu.VMEM_SHARED((TABLE,), jnp.float32),
        pltpu.VMEM((CH,), jnp.float32), pltpu.VMEM((CH,), jnp.float32),
        pltpu.VMEM((CH,), jnp.int32), pltpu.VMEM((CH,), jnp.int32),
        pltpu.VMEM((CH,), jnp.float32), pltpu.VMEM((CH,), jnp.float32),
        pltpu.SemaphoreType.DMA, pltpu.SemaphoreType.DMA,
        pltpu.SemaphoreType.DMA, pltpu.SemaphoreType.DMA,
        pltpu.SemaphoreType.DMA, pltpu.SemaphoreType.DMA,
        pltpu.SemaphoreType.DMA, pltpu.SemaphoreType.DMA,
    ],
)
def _lookup(soft_hbm, table_hbm, out_hbm, spt, soft0, soft1, idx0, idx1,
            got0, got1, lsem0, lsem1, gsem0, gsem1, ssem0, ssem1, stsem,
            bsem):
    soft = (soft0, soft1)
    idx = (idx0, idx1)
    got = (got0, got1)
    lsem = (lsem0, lsem1)
    gsem = (gsem0, gsem1)
    ssem = (ssem0, ssem1)

    sid = lax.axis_index("s")
    wid = sid * NC + lax.axis_index("c")
    base = wid * PER_W

    n_rounds = (NSTAGE + NS - 1) // NS

    def st_off(j):
        return j * NS * CH + sid * CH

    def stA(j):
        return pltpu.make_async_copy(
            table_hbm.at[pl.ds(st_off(j), CH)], got[j & 1], stsem)

    def stB(j):
        return pltpu.make_async_copy(
            got[j & 1], spt.at[pl.ds(st_off(j), CH)], bsem)

    def load(c, s):
        return pltpu.make_async_copy(
            soft_hbm.at[pl.ds(base + c * CH, CH)], soft[s], lsem[s])

    def gather(s):
        return pltpu.make_async_copy(spt.at[idx[s]], got[s], gsem[s])

    def store(c, s):
        return pltpu.make_async_copy(
            got[s], out_hbm.at[pl.ds(base + c * CH, CH)], ssem[s])

    def quant_span(s, lo, hi):
        src, dst = soft[s], idx[s]

        @plsc.parallel_loop(lo, hi, step=L, unroll=8)
        def _body(i):
            v = src[pl.ds(i, L)]
            v = jnp.minimum(jnp.maximum(v, -0.999), 0.999)
            dst[pl.ds(i, L)] = (
                (v + 1.0) / 2.0 * float(TABLE)).astype(jnp.int32)

    def quant(s):
        quant_span(s, 0, CH)

    QB = (CH // L) // 4

    def quant_block(b):
        s = b // 4
        base_i = (b % 4) * QB
        quant_span(s, base_i * L, (base_i + QB) * L)

    load(0, 0).start()
    load(1, 1).start()

    def pred(j):
        return st_off(j) + CH <= TABLE

    @pl.when(pred(0))
    def _():
        stA(0).start()
    @pl.when(pred(1))
    def _():
        stA(1).start()
    load(0, 0).wait()
    for j in range(n_rounds):
        if j >= 2:
            @pl.when(pred(j - 2))
            def _(j=j):
                stB(j - 2).wait()
            @pl.when(pred(j))
            def _(j=j):
                stA(j).start()
        if j == 4:
            load(1, 1).wait()
        quant_block(j)
        @pl.when(pred(j))
        def _(j=j):
            stA(j).wait()
            stB(j).start()
    @pl.when(pred(n_rounds - 2))
    def _():
        stB(n_rounds - 2).wait()
    @pl.when(pred(n_rounds - 1))
    def _():
        stB(n_rounds - 1).wait()
    @pl.when(sid == NS - 1)
    def _():
        pltpu.sync_copy(table_hbm.at[pl.ds(NSTAGE * CH, STAGE_TAIL)],
                        got0.at[pl.ds(0, STAGE_TAIL)])
        pltpu.sync_copy(got0.at[pl.ds(0, STAGE_TAIL)],
                        spt.at[pl.ds(NSTAGE * CH, STAGE_TAIL)])
    plsc.subcore_barrier()

    for c in range(NCH):
        s = c & 1
        p = s ^ 1
        if c >= 2:
            load(c, s).wait()
            quant(s)
        if c + 2 < NCH:
            load(c + 2, s).start()
        if c >= 2:
            store(c - 2, s).wait()
        gather(s).start()
        if c >= 1:
            gather(p).wait()
            store(c - 1, p).start()
    last = NCH - 1
    sl = last & 1
    gather(sl).wait()
    store(last, sl).start()
    store(last - 1, sl ^ 1).wait()
    store(last, sl).wait()


def kernel(x, pallette, indices):
    out = _lookup(indices.reshape(-1), pallette.reshape(-1))
    return out.reshape(ROWS, COLS)


# speedup vs baseline: 1.0231x; 1.0146x over previous
import functools

import jax
import jax.numpy as jnp
from jax import lax
from jax.experimental import pallas as pl
from jax.experimental.pallas import tpu as pltpu
from jax.experimental.pallas import tpu_sc as plsc

ROWS, COLS = 16384, 128
N = ROWS * COLS
TABLE = 1000000
NC, NS, L = 2, 16, 16
NW = NC * NS
PER_W = N // NW
CH = 8192
NCH = PER_W // CH
NSTAGE = TABLE // CH
STAGE_TAIL = TABLE - NSTAGE * CH

_mesh = plsc.VectorSubcoreMesh(core_axis_name="c", subcore_axis_name="s")


@functools.partial(
    pl.kernel,
    mesh=_mesh,
    out_type=jax.ShapeDtypeStruct((N,), jnp.float32),
    scratch_types=[
        pltpu.VMEM_SHARED((TABLE,), jnp.float32),
        pltpu.VMEM((CH,), jnp.float32), pltpu.VMEM((CH,), jnp.float32),
        pltpu.VMEM((CH,), jnp.int32), pltpu.VMEM((CH,), jnp.int32),
        pltpu.VMEM((CH,), jnp.float32), pltpu.VMEM((CH,), jnp.float32),
        pltpu.VMEM((CH,), jnp.float32),
        pltpu.SemaphoreType.DMA, pltpu.SemaphoreType.DMA,
        pltpu.SemaphoreType.DMA, pltpu.SemaphoreType.DMA,
        pltpu.SemaphoreType.DMA, pltpu.SemaphoreType.DMA,
        pltpu.SemaphoreType.DMA, pltpu.SemaphoreType.DMA,
    ],
)
def _lookup(soft_hbm, table_hbm, out_hbm, spt, soft0, soft1, idx0, idx1,
            got0, got1, bnc, lsem0, lsem1, gsem0, gsem1, ssem0, ssem1,
            stsem, bsem):
    soft = (soft0, soft1)
    idx = (idx0, idx1)
    got = (got0, got1)
    lsem = (lsem0, lsem1)
    gsem = (gsem0, gsem1)
    ssem = (ssem0, ssem1)

    sid = lax.axis_index("s")
    wid = sid * NC + lax.axis_index("c")
    base = wid * PER_W

    n_rounds = (NSTAGE + NS - 1) // NS

    def st_off(j):
        return j * NS * CH + sid * CH

    bounce = (got0, got1, bnc)

    def stA(j):
        return pltpu.make_async_copy(
            table_hbm.at[pl.ds(st_off(j), CH)], bounce[j % 3], stsem)

    def stB(j):
        return pltpu.make_async_copy(
            bounce[j % 3], spt.at[pl.ds(st_off(j), CH)], bsem)

    def load(c, s):
        return pltpu.make_async_copy(
            soft_hbm.at[pl.ds(base + c * CH, CH)], soft[s], lsem[s])

    def gather(s):
        return pltpu.make_async_copy(spt.at[idx[s]], got[s], gsem[s])

    def store(c, s):
        return pltpu.make_async_copy(
            got[s], out_hbm.at[pl.ds(base + c * CH, CH)], ssem[s])

    def quant_span(s, lo, hi):
        src, dst = soft[s], idx[s]

        @plsc.parallel_loop(lo, hi, step=L, unroll=8)
        def _body(i):
            v = src[pl.ds(i, L)]
            v = jnp.minimum(jnp.maximum(v, -0.999), 0.999)
            dst[pl.ds(i, L)] = (
                (v + 1.0) / 2.0 * float(TABLE)).astype(jnp.int32)

    def quant(s):
        quant_span(s, 0, CH)

    QB = (CH // L) // 4

    def quant_block(b):
        s = b // 4
        base_i = (b % 4) * QB
        quant_span(s, base_i * L, (base_i + QB) * L)

    load(0, 0).start()
    load(1, 1).start()

    def pred(j):
        return st_off(j) + CH <= TABLE

    for jj in range(3):
        @pl.when(pred(jj))
        def _(jj=jj):
            stA(jj).start()
    load(0, 0).wait()
    for j in range(n_rounds):
        if j >= 3:
            @pl.when(pred(j - 3))
            def _(j=j):
                stB(j - 3).wait()
            @pl.when(pred(j))
            def _(j=j):
                stA(j).start()
        if j == 4:
            load(1, 1).wait()
        quant_block(j)
        @pl.when(pred(j))
        def _(j=j):
            stA(j).wait()
            stB(j).start()
    for jj in range(n_rounds - 3, n_rounds):
        @pl.when(pred(jj))
        def _(jj=jj):
            stB(jj).wait()
    @pl.when(sid == NS - 1)
    def _():
        pltpu.sync_copy(table_hbm.at[pl.ds(NSTAGE * CH, STAGE_TAIL)],
                        got0.at[pl.ds(0, STAGE_TAIL)])
        pltpu.sync_copy(got0.at[pl.ds(0, STAGE_TAIL)],
                        spt.at[pl.ds(NSTAGE * CH, STAGE_TAIL)])
    plsc.subcore_barrier()

    for c in range(NCH):
        s = c & 1
        p = s ^ 1
        if c >= 2:
            load(c, s).wait()
            quant(s)
        if c + 2 < NCH:
            load(c + 2, s).start()
        if c >= 2:
            store(c - 2, s).wait()
        gather(s).start()
        if c >= 1:
            gather(p).wait()
            store(c - 1, p).start()
    last = NCH - 1
    sl = last & 1
    gather(sl).wait()
    store(last, sl).start()
    store(last - 1, sl ^ 1).wait()
    store(last, sl).wait()


def kernel(x, pallette, indices):
    out = _lookup(indices.reshape(-1), pallette.reshape(-1))
    return out.reshape(ROWS, COLS)


# speedup vs baseline: 1.0346x; 1.0113x over previous
import functools

import jax
import jax.numpy as jnp
from jax import lax
from jax.experimental import pallas as pl
from jax.experimental.pallas import tpu as pltpu
from jax.experimental.pallas import tpu_sc as plsc

ROWS, COLS = 16384, 128
N = ROWS * COLS
TABLE = 1000000
NC, NS, L = 2, 16, 16
NW = NC * NS
PER_W = N // NW
CH = 8192
NCH = PER_W // CH
NSTAGE = TABLE // CH
STAGE_TAIL = TABLE - NSTAGE * CH

_mesh = plsc.VectorSubcoreMesh(core_axis_name="c", subcore_axis_name="s")


@functools.partial(
    pl.kernel,
    mesh=_mesh,
    out_type=jax.ShapeDtypeStruct((N,), jnp.float32),
    scratch_types=[
        pltpu.VMEM_SHARED((TABLE,), jnp.float32),
        pltpu.VMEM((CH,), jnp.float32), pltpu.VMEM((CH,), jnp.float32),
        pltpu.VMEM((CH,), jnp.int32), pltpu.VMEM((CH,), jnp.int32),
        pltpu.VMEM((CH,), jnp.float32), pltpu.VMEM((CH,), jnp.float32),
        pltpu.VMEM((CH,), jnp.float32), pltpu.VMEM((CH,), jnp.float32),
        pltpu.SemaphoreType.DMA, pltpu.SemaphoreType.DMA,
        pltpu.SemaphoreType.DMA, pltpu.SemaphoreType.DMA,
        pltpu.SemaphoreType.DMA, pltpu.SemaphoreType.DMA,
        pltpu.SemaphoreType.DMA, pltpu.SemaphoreType.DMA,
    ],
)
def _lookup(soft_hbm, table_hbm, out_hbm, spt, soft0, soft1, idx0, idx1,
            got0, got1, bnc, bnc2, lsem0, lsem1, gsem0, gsem1, ssem0,
            ssem1, stsem, bsem):
    soft = (soft0, soft1)
    idx = (idx0, idx1)
    got = (got0, got1)
    lsem = (lsem0, lsem1)
    gsem = (gsem0, gsem1)
    ssem = (ssem0, ssem1)

    sid = lax.axis_index("s")
    wid = sid * NC + lax.axis_index("c")
    base = wid * PER_W

    n_rounds = (NSTAGE + NS - 1) // NS

    def st_off(j):
        return j * NS * CH + sid * CH

    bounce = (got0, got1, bnc, bnc2)
    NB = len(bounce)

    def stA(j):
        return pltpu.make_async_copy(
            table_hbm.at[pl.ds(st_off(j), CH)], bounce[j % NB], stsem)

    def stB(j):
        return pltpu.make_async_copy(
            bounce[j % NB], spt.at[pl.ds(st_off(j), CH)], bsem)

    def load(c, s):
        return pltpu.make_async_copy(
            soft_hbm.at[pl.ds(base + c * CH, CH)], soft[s], lsem[s])

    def gather(s):
        return pltpu.make_async_copy(spt.at[idx[s]], got[s], gsem[s])

    def store(c, s):
        return pltpu.make_async_copy(
            got[s], out_hbm.at[pl.ds(base + c * CH, CH)], ssem[s])

    def quant_span(s, lo, hi):
        src, dst = soft[s], idx[s]

        @plsc.parallel_loop(lo, hi, step=L, unroll=8)
        def _body(i):
            v = src[pl.ds(i, L)]
            v = jnp.minimum(jnp.maximum(v, -0.999), 0.999)
            dst[pl.ds(i, L)] = (
                (v + 1.0) / 2.0 * float(TABLE)).astype(jnp.int32)

    def quant(s):
        quant_span(s, 0, CH)

    QB = (CH // L) // 4

    def quant_block(b):
        s = b // 4
        base_i = (b % 4) * QB
        quant_span(s, base_i * L, (base_i + QB) * L)

    load(0, 0).start()
    load(1, 1).start()

    def pred(j):
        return st_off(j) + CH <= TABLE

    for jj in range(NB):
        @pl.when(pred(jj))
        def _(jj=jj):
            stA(jj).start()
    load(0, 0).wait()
    for j in range(n_rounds):
        if j >= NB:
            @pl.when(pred(j - NB))
            def _(j=j):
                stB(j - NB).wait()
            @pl.when(pred(j))
            def _(j=j):
                stA(j).start()
        if j == 4:
            load(1, 1).wait()
        quant_block(j)
        @pl.when(pred(j))
        def _(j=j):
            stA(j).wait()
            stB(j).start()
    for jj in range(n_rounds - NB, n_rounds):
        @pl.when(pred(jj))
        def _(jj=jj):
            stB(jj).wait()
    @pl.when(sid == NS - 1)
    def _():
        pltpu.sync_copy(table_hbm.at[pl.ds(NSTAGE * CH, STAGE_TAIL)],
                        got0.at[pl.ds(0, STAGE_TAIL)])
        pltpu.sync_copy(got0.at[pl.ds(0, STAGE_TAIL)],
                        spt.at[pl.ds(NSTAGE * CH, STAGE_TAIL)])
    plsc.subcore_barrier()

    for c in range(NCH):
        s = c & 1
        p = s ^ 1
        if c >= 2:
            load(c, s).wait()
            quant(s)
        if c + 2 < NCH:
            load(c + 2, s).start()
        if c >= 2:
            store(c - 2, s).wait()
        gather(s).start()
        if c >= 1:
            gather(p).wait()
            store(c - 1, p).start()
    last = NCH - 1
    sl = last & 1
    gather(sl).wait()
    store(last, sl).start()
    store(last - 1, sl ^ 1).wait()
    store(last, sl).wait()


def kernel(x, pallette, indices):
    out = _lookup(indices.reshape(-1), pallette.reshape(-1))
    return out.reshape(ROWS, COLS)
